# baseline (speedup 1.0000x reference)
import dataclasses
import functools

import jax
import jax.numpy as jnp
from jax import lax
from jax.experimental import pallas as pl
from jax.experimental.pallas import tpu as pltpu
from jax.experimental.pallas import tpu_sc as plsc

NC = 2
NS = 16
L = 16
NW = NC * NS

_MAGIC = 8388608.0


def _build_sc_gather(n_boxes, g, d2):
    boxes_per_w = n_boxes // NW
    flat_per_w = 4 * boxes_per_w
    CH = 128
    NBUF = 3
    n_chunks = boxes_per_w // CH
    n_vec = boxes_per_w // L

    mesh = plsc.VectorSubcoreMesh(core_axis_name="c", subcore_axis_name="s")
    cp = pltpu.CompilerParams()
    if "needs_layout_passes" in pltpu.CompilerParams.__dataclass_fields__:
        cp = dataclasses.replace(cp, needs_layout_passes=False)

    @functools.partial(
        pl.kernel,
        mesh=mesh,
        compiler_params=cp,
        out_type=jax.ShapeDtypeStruct((n_boxes, d2), jnp.float32),
        scratch_types=(
            [pltpu.VMEM((flat_per_w,), jnp.float32),
             pltpu.VMEM((L,), jnp.float32),
             pltpu.VMEM((boxes_per_w,), jnp.int32)]
            + [pltpu.VMEM((CH, d2), jnp.float32) for _ in range(NBUF)]
            + [pltpu.SemaphoreType.DMA for _ in range(2 * NBUF)]
        ),
    )
    def sc_kernel(boxes_hbm, scale_hbm, table_hbm, out_hbm,
                  bx_v, scale_v, idx_v, *bufs_and_sems):
        bufs = bufs_and_sems[:NBUF]
        gsems = bufs_and_sems[NBUF:2 * NBUF]
        osems = bufs_and_sems[2 * NBUF:3 * NBUF]

        wid = lax.axis_index("s") * NC + lax.axis_index("c")
        pltpu.sync_copy(boxes_hbm.at[pl.ds(wid * flat_per_w, flat_per_w)], bx_v)
        pltpu.sync_copy(scale_hbm, scale_v)
        scale = scale_v[...]

        jvec = lax.iota(jnp.int32, L)
        fx = 4 * jvec
        fy = 4 * jvec + 1

        @pl.loop(0, n_vec)
        def _(t):
            base = (4 * L) * t
            xv = plsc.load_gather(bx_v, [base + fx])
            yv = plsc.load_gather(bx_v, [base + fy])
            iw = ((xv * scale + _MAGIC) - _MAGIC).astype(jnp.int32)
            ih = ((yv * scale + _MAGIC) - _MAGIC).astype(jnp.int32)
            idx_v[pl.ds(t * L, L)] = ih * g + iw

        bbase = wid * boxes_per_w
        gd = [None] * NBUF
        od = [None] * NBUF

        def start_gather(c):
            b = c % NBUF
            gd[b] = pltpu.async_copy(
                table_hbm.at[idx_v.at[pl.ds(c * CH, CH)]], bufs[b], gsems[b])

        LK = NBUF - 1
        for c in range(min(LK, n_chunks)):
            start_gather(c)
        for c in range(n_chunks):
            b = c % NBUF
            gd[b].wait()
            if od[b] is not None:
                od[b].wait()
            od[b] = pltpu.async_copy(
                bufs[b], out_hbm.at[pl.ds(bbase + c * CH, CH)], osems[b])
            nxt = c + LK
            if nxt < n_chunks:
                bb = nxt % NBUF
                if od[bb] is not None:
                    od[bb].wait()
                    od[bb] = None
                start_gather(nxt)
        for x in od:
            if x is not None:
                x.wait()

    return sc_kernel


def kernel(boxes_norm, grid_size, h_table, w_table):
    n, _ = boxes_norm.shape
    g, d = h_table.shape
    combo = jnp.concatenate([
        jnp.broadcast_to(h_table[:, None, :], (g, g, d)),
        jnp.broadcast_to(w_table[None, :, :], (g, g, d)),
    ], axis=-1).reshape(g * g, 2 * d)
    scale = jnp.full((L,), (grid_size - 1), dtype=jnp.float32)
    boxes_flat = boxes_norm.reshape(-1)
    sc = _build_sc_gather(n, g, 2 * d)
    return sc(boxes_flat, scale, combo).reshape(n, 2 * d, 1, 1)


# speedup vs baseline: 1.2821x; 1.2821x over previous
import dataclasses
import functools

import jax
import jax.numpy as jnp
from jax import lax
from jax.experimental import pallas as pl
from jax.experimental.pallas import tpu as pltpu
from jax.experimental.pallas import tpu_sc as plsc

NC = 2
NS = 16
L = 16
NW = NC * NS

_MAGIC = 8388608.0


def _build_sc_kernel(n_boxes, g, d):
    rows_total = 2 * n_boxes
    rows_per_w = rows_total // NW
    boxes_per_w = n_boxes // NW
    flat_per_w = 4 * boxes_per_w
    CH = 256
    NBUF = 3
    n_chunks = rows_per_w // CH
    n_vec = rows_per_w // L
    gg = g * g
    rep_rows = 2 * gg
    hw_per_w = rep_rows // 2 // NS
    vpr = d // L

    mesh = plsc.VectorSubcoreMesh(core_axis_name="c", subcore_axis_name="s")
    cp = pltpu.CompilerParams()
    if "needs_layout_passes" in pltpu.CompilerParams.__dataclass_fields__:
        cp = dataclasses.replace(cp, needs_layout_passes=False)

    @functools.partial(
        pl.kernel,
        mesh=mesh,
        compiler_params=cp,
        out_type=jax.ShapeDtypeStruct((rows_total, d), jnp.float32),
        scratch_types=(
            [pltpu.HBM((NC * rep_rows, d), jnp.float32),
             pltpu.VMEM((flat_per_w,), jnp.float32),
             pltpu.VMEM((L,), jnp.float32),
             pltpu.VMEM((rows_per_w,), jnp.int32),
             pltpu.VMEM((g, d), jnp.float32),
             pltpu.SemaphoreType.DMA]
            + [pltpu.VMEM((CH, d), jnp.float32) for _ in range(NBUF)]
            + [pltpu.SemaphoreType.DMA for _ in range(2 * NBUF)]
        ),
    )
    def sc_kernel(boxes_hbm, scale_hbm, htab_hbm, wtab_hbm, out_hbm,
                  rep_hbm, bx_v, scale_v, idx_v, tab_v, tsem,
                  *bufs_and_sems):
        bufs = bufs_and_sems[:NBUF]
        gsems = bufs_and_sems[NBUF:2 * NBUF]
        osems = bufs_and_sems[2 * NBUF:3 * NBUF]

        sid = lax.axis_index("c")
        s = lax.axis_index("s")
        wid = s * NC + sid
        sc_base = sid * rep_rows

        bxd = pltpu.async_copy(
            boxes_hbm.at[pl.ds(wid * flat_per_w, flat_per_w)], bx_v, tsem)

        pltpu.sync_copy(wtab_hbm, tab_v)
        w0 = sc_base + gg + 2 * g * s
        cw0 = pltpu.async_copy(tab_v, rep_hbm.at[pl.ds(w0, g)], osems[0])
        cw1 = pltpu.async_copy(tab_v, rep_hbm.at[pl.ds(w0 + g, g)], osems[1])
        bxd.wait()
        cw0.wait()
        cw1.wait()

        @pl.loop(0, 2)
        def _(r):
            hrow = 2 * s + r
            pltpu.sync_copy(htab_hbm.at[pl.ds(hrow, 1)], tab_v.at[pl.ds(0, 1)])
            src = tab_v.at[0]
            @pl.loop(1, g)
            def _(i):
                dstrow = tab_v.at[i]
                @pl.loop(0, vpr)
                def _(c):
                    dstrow[pl.ds(c * L, L)] = src[pl.ds(c * L, L)]
            pltpu.sync_copy(
                tab_v, rep_hbm.at[pl.ds(sc_base + 2 * g * s + r * g, g)])

        pltpu.sync_copy(scale_hbm, scale_v)
        scale = scale_v[...]
        jvec = lax.iota(jnp.int32, L)
        half4 = 4 * lax.shift_right_logical(jvec, 1)
        parity = lax.bitwise_and(jvec, 1)
        poff = parity * gg + sc_base

        @pl.loop(0, n_vec)
        def _(t):
            base = (2 * L) * t
            xv = plsc.load_gather(bx_v, [base + half4])
            yv = plsc.load_gather(bx_v, [base + half4 + 1])
            iw = ((xv * scale + _MAGIC) - _MAGIC).astype(jnp.int32)
            ih = ((yv * scale + _MAGIC) - _MAGIC).astype(jnp.int32)
            idx_v[pl.ds(t * L, L)] = ih * g + iw + poff

        plsc.subcore_barrier()

        wbase = wid * rows_per_w
        gd = [None] * NBUF
        od = [None] * NBUF

        def start_gather(c):
            b = c % NBUF
            gd[b] = pltpu.async_copy(
                rep_hbm.at[idx_v.at[pl.ds(c * CH, CH)]], bufs[b], gsems[b])

        LK = NBUF - 1
        for c in range(min(LK, n_chunks)):
            start_gather(c)
        for c in range(n_chunks):
            b = c % NBUF
            gd[b].wait()
            if od[b] is not None:
                od[b].wait()
            od[b] = pltpu.async_copy(
                bufs[b], out_hbm.at[pl.ds(wbase + c * CH, CH)], osems[b])
            nxt = c + LK
            if nxt < n_chunks:
                bb = nxt % NBUF
                if od[bb] is not None:
                    od[bb].wait()
                    od[bb] = None
                start_gather(nxt)
        for x in od:
            if x is not None:
                x.wait()

    return sc_kernel


def kernel(boxes_norm, grid_size, h_table, w_table):
    n, _ = boxes_norm.shape
    g, d = h_table.shape
    scale = jnp.full((L,), (grid_size - 1), dtype=jnp.float32)
    boxes_flat = boxes_norm.reshape(-1)
    sc = _build_sc_kernel(n, g, d)
    out = sc(boxes_flat, scale, h_table, w_table)
    return out.reshape(n, 2 * d, 1, 1)


# speedup vs baseline: 1.3992x; 1.0913x over previous
import dataclasses
import functools

import jax
import jax.numpy as jnp
from jax import lax
from jax.experimental import pallas as pl
from jax.experimental.pallas import tpu as pltpu
from jax.experimental.pallas import tpu_sc as plsc

NC = 2
NS = 16
L = 16
NW = NC * NS

_MAGIC = 8388608.0


def _build_sc_kernel(n_boxes, g, d):
    rows_total = 2 * n_boxes
    rows_per_w = rows_total // NW
    boxes_per_w = n_boxes // NW
    flat_per_w = 4 * boxes_per_w
    CH = 128
    NBUF = 3
    n_chunks = rows_per_w // CH
    n_vec = rows_per_w // L
    gg = g * g
    rep_rows = 2 * gg
    hw_per_w = rep_rows // 2 // NS
    vpr = d // L

    mesh = plsc.VectorSubcoreMesh(core_axis_name="c", subcore_axis_name="s")
    cp = pltpu.CompilerParams()
    if "needs_layout_passes" in pltpu.CompilerParams.__dataclass_fields__:
        cp = dataclasses.replace(cp, needs_layout_passes=False)

    @functools.partial(
        pl.kernel,
        mesh=mesh,
        compiler_params=cp,
        out_type=jax.ShapeDtypeStruct((rows_total, d), jnp.float32),
        scratch_types=(
            [pltpu.HBM((NC * rep_rows, d), jnp.float32),
             pltpu.VMEM((boxes_per_w, 4), jnp.float32),
             pltpu.VMEM((L,), jnp.float32),
             pltpu.VMEM((rows_per_w,), jnp.int32),
             pltpu.VMEM((g, d), jnp.float32),
             pltpu.SemaphoreType.DMA]
            + [pltpu.VMEM((CH, d), jnp.float32) for _ in range(NBUF)]
            + [pltpu.SemaphoreType.DMA for _ in range(2 * NBUF)]
        ),
    )
    def sc_kernel(boxes_hbm, scale_hbm, htab_hbm, wtab_hbm, out_hbm,
                  rep_hbm, bx_v, scale_v, idx_v, tab_v, tsem,
                  *bufs_and_sems):
        bufs = bufs_and_sems[:NBUF]
        gsems = bufs_and_sems[NBUF:2 * NBUF]
        osems = bufs_and_sems[2 * NBUF:3 * NBUF]

        sid = lax.axis_index("c")
        s = lax.axis_index("s")
        wid = s * NC + sid
        sc_base = sid * rep_rows

        bxd = pltpu.async_copy(
            boxes_hbm.at[pl.ds(wid * boxes_per_w, boxes_per_w)], bx_v, tsem)

        pltpu.sync_copy(wtab_hbm, tab_v)
        w0 = sc_base + gg + 2 * g * s
        cw0 = pltpu.async_copy(tab_v, rep_hbm.at[pl.ds(w0, g)], osems[0])
        cw1 = pltpu.async_copy(tab_v, rep_hbm.at[pl.ds(w0 + g, g)], osems[1])
        bxd.wait()
        cw0.wait()
        cw1.wait()

        @pl.loop(0, 2)
        def _(r):
            hrow = 2 * s + r
            pltpu.sync_copy(htab_hbm.at[pl.ds(hrow, 1)], tab_v.at[pl.ds(0, 1)])
            src = tab_v.at[0]
            @pl.loop(1, g)
            def _(i):
                dstrow = tab_v.at[i]
                @pl.loop(0, vpr)
                def _(c):
                    dstrow[pl.ds(c * L, L)] = src[pl.ds(c * L, L)]
            pltpu.sync_copy(
                tab_v, rep_hbm.at[pl.ds(sc_base + 2 * g * s + r * g, g)])

        pltpu.sync_copy(scale_hbm, scale_v)
        scale = scale_v[...]
        jvec = lax.iota(jnp.int32, L)
        half = lax.shift_right_logical(jvec, 1)
        parity = lax.bitwise_and(jvec, 1)
        poff = parity * gg + sc_base
        col0 = jnp.zeros((L,), jnp.int32)
        col1 = col0 + 1

        @pl.loop(0, n_vec)
        def _(t):
            rows = (L // 2) * t + half
            xv = plsc.load_gather(bx_v, [rows, col0])
            yv = plsc.load_gather(bx_v, [rows, col1])
            iw = ((xv * scale + _MAGIC) - _MAGIC).astype(jnp.int32)
            ih = ((yv * scale + _MAGIC) - _MAGIC).astype(jnp.int32)
            idx_v[pl.ds(t * L, L)] = ih * g + iw + poff

        plsc.subcore_barrier()

        wbase = wid * rows_per_w
        gd = [None] * NBUF
        od = [None] * NBUF

        def start_gather(c):
            b = c % NBUF
            gd[b] = pltpu.async_copy(
                rep_hbm.at[idx_v.at[pl.ds(c * CH, CH)]], bufs[b], gsems[b])

        LK = NBUF - 1
        for c in range(min(LK, n_chunks)):
            start_gather(c)
        for c in range(n_chunks):
            b = c % NBUF
            gd[b].wait()
            if od[b] is not None:
                od[b].wait()
            od[b] = pltpu.async_copy(
                bufs[b], out_hbm.at[pl.ds(wbase + c * CH, CH)], osems[b])
            nxt = c + LK
            if nxt < n_chunks:
                bb = nxt % NBUF
                if od[bb] is not None:
                    od[bb].wait()
                    od[bb] = None
                start_gather(nxt)
        for x in od:
            if x is not None:
                x.wait()

    return sc_kernel


def kernel(boxes_norm, grid_size, h_table, w_table):
    n, _ = boxes_norm.shape
    g, d = h_table.shape
    scale = jnp.full((L,), (grid_size - 1), dtype=jnp.float32)
    sc = _build_sc_kernel(n, g, d)
    out = sc(boxes_norm, scale, h_table, w_table)
    return out.reshape(n, 2 * d, 1, 1)
